# baseline (speedup 1.0000x reference)
import jax
import jax.numpy as jnp
from jax import lax
from jax.experimental import pallas as pl
from jax.experimental.pallas import tpu as pltpu
from jax.experimental.pallas import tpu_sc as plsc

B, C = 16384, 1000
NW = 32
RPW = B // NW
CH = 32
NCHUNK = RPW // CH
GPC = CH // 16
NLOAD = C // 16
TAIL0 = C - 16


def _cw_body(pred_hbm, y_hbm, out_hbm, buf0, buf1, y_v, out_v, scr,
             sem0, sem1):
    cid = lax.axis_index("c")
    sid = lax.axis_index("s")
    wid = sid * 2 + cid
    row0 = wid * RPW
    pltpu.sync_copy(y_hbm.at[pl.ds(row0, RPW)], y_v)

    bufs = [buf0, buf1]
    sems = [sem0, sem1]
    neg_inf = jnp.full((16,), -jnp.inf, jnp.float32)
    lane = lax.iota(jnp.int32, 16)
    last = jnp.full((16,), 15, jnp.int32)

    pltpu.async_copy(pred_hbm.at[pl.ds(row0, CH)], bufs[0], sems[0])
    pltpu.async_copy(pred_hbm.at[pl.ds(row0 + CH, CH)], bufs[1], sems[1])

    def chunk_pair(cc, carry):
        for b in range(2):
            ci = cc * 2 + b
            buf, sem = bufs[b], sems[b]
            pltpu.make_async_copy(
                pred_hbm.at[pl.ds(row0, CH)], buf, sem).wait()
            for g in range(GPC):
                goff = ci * CH + g * 16
                yv = y_v[pl.ds(pl.multiple_of(goff, 16), 16)]
                rowv = g * 16 + lane
                class_pred = plsc.load_gather(buf, [rowv, yv])
                plsc.store_scatter(buf, [rowv, yv], neg_inf)

                @plsc.parallel_loop(0, 4)
                def row_quad(i):
                    for u in range(4):
                        r = g * 16 + i * 4 + u
                        accs = [neg_inf, neg_inf, neg_inf, neg_inf]
                        for k in range(NLOAD):
                            accs[k % 4] = jnp.maximum(
                                accs[k % 4], buf[r, pl.ds(16 * k, 16)])
                        accs[2] = jnp.maximum(
                            accs[2], buf[r, pl.ds(TAIL0, 16)])
                        comb = jnp.maximum(jnp.maximum(accs[0], accs[1]),
                                           jnp.maximum(accs[2], accs[3]))
                        scr[i * 4 + u, pl.ds(0, 16)] = plsc.cummax(comb)
                rowmax = plsc.load_gather(scr, [lane, last])
                out_v[pl.ds(pl.multiple_of(goff, 16), 16)] = (
                    rowmax - class_pred)

            @pl.when(ci + 2 < NCHUNK)
            def _():
                nxt = row0 + pl.multiple_of((ci + 2) * CH, CH)
                pltpu.async_copy(
                    pred_hbm.at[pl.ds(nxt, CH)], buf, sem)

        return carry

    lax.fori_loop(0, NCHUNK // 2, chunk_pair, 0)
    pltpu.sync_copy(out_v, out_hbm.at[pl.ds(row0, RPW)])


_run = pl.kernel(
    _cw_body,
    out_type=jax.ShapeDtypeStruct((B,), jnp.float32),
    mesh=plsc.VectorSubcoreMesh(core_axis_name="c", subcore_axis_name="s"),
    scratch_types=[
        pltpu.VMEM((CH, C), jnp.float32),
        pltpu.VMEM((CH, C), jnp.float32),
        pltpu.VMEM((RPW,), jnp.int32),
        pltpu.VMEM((RPW,), jnp.float32),
        pltpu.VMEM((16, 16), jnp.float32),
        pltpu.SemaphoreType.DMA,
        pltpu.SemaphoreType.DMA,
    ],
    compiler_params=pltpu.CompilerParams(needs_layout_passes=False),
)


@jax.jit
def kernel(pred, y):
    return _run(pred, y.astype(jnp.int32))


# speedup vs baseline: 1.0956x; 1.0956x over previous
import jax
import jax.numpy as jnp
from jax import lax
from jax.experimental import pallas as pl
from jax.experimental.pallas import tpu as pltpu
from jax.experimental.pallas import tpu_sc as plsc

B, C = 16384, 1000
NW = 32
RPW = B // NW
CH = 32
NCHUNK = RPW // CH
GPC = CH // 16
NLOAD = C // 16
TAIL0 = C - 16


def _cw_body(pred_hbm, y_hbm, out_hbm, buf0, buf1, y_v, out_v, scr,
             sem0, sem1):
    cid = lax.axis_index("c")
    sid = lax.axis_index("s")
    wid = sid * 2 + cid
    row0 = wid * RPW
    pltpu.sync_copy(y_hbm.at[pl.ds(row0, RPW)], y_v)

    bufs = [buf0, buf1]
    sems = [sem0, sem1]
    neg_inf = jnp.full((16,), -jnp.inf, jnp.float32)
    lane = lax.iota(jnp.int32, 16)
    last = jnp.full((16,), 15, jnp.int32)

    pltpu.async_copy(pred_hbm.at[pl.ds(row0, CH)], bufs[0], sems[0])
    pltpu.async_copy(pred_hbm.at[pl.ds(row0 + CH, CH)], bufs[1], sems[1])

    def chunk_pair(cc, carry):
        for b in range(2):
            ci = cc * 2 + b
            buf, sem = bufs[b], sems[b]
            pltpu.make_async_copy(
                pred_hbm.at[pl.ds(row0, CH)], buf, sem).wait()
            for g in range(GPC):
                goff = ci * CH + g * 16
                yv = y_v[pl.ds(pl.multiple_of(goff, 16), 16)]
                rowv = g * 16 + lane
                class_pred = plsc.load_gather(buf, [rowv, yv])
                plsc.store_scatter(buf, [rowv, yv], neg_inf)

                def row_quad(i, carry):
                    for u in range(4):
                        r = g * 16 + i * 4 + u
                        accs = [neg_inf, neg_inf, neg_inf, neg_inf]
                        for k in range(NLOAD):
                            accs[k % 4] = jnp.maximum(
                                accs[k % 4], buf[r, pl.ds(16 * k, 16)])
                        accs[2] = jnp.maximum(
                            accs[2], buf[r, pl.ds(TAIL0, 16)])
                        comb = jnp.maximum(jnp.maximum(accs[0], accs[1]),
                                           jnp.maximum(accs[2], accs[3]))
                        scr[i * 4 + u, pl.ds(0, 16)] = plsc.cummax(comb)
                    return carry

                lax.fori_loop(0, 4, row_quad, 0)
                rowmax = plsc.load_gather(scr, [lane, last])
                out_v[pl.ds(pl.multiple_of(goff, 16), 16)] = (
                    rowmax - class_pred)

            @pl.when(ci + 2 < NCHUNK)
            def _():
                nxt = row0 + pl.multiple_of((ci + 2) * CH, CH)
                pltpu.async_copy(
                    pred_hbm.at[pl.ds(nxt, CH)], buf, sem)

        return carry

    lax.fori_loop(0, NCHUNK // 2, chunk_pair, 0)
    pltpu.sync_copy(out_v, out_hbm.at[pl.ds(row0, RPW)])


_run = pl.kernel(
    _cw_body,
    out_type=jax.ShapeDtypeStruct((B,), jnp.float32),
    mesh=plsc.VectorSubcoreMesh(core_axis_name="c", subcore_axis_name="s"),
    scratch_types=[
        pltpu.VMEM((CH, C), jnp.float32),
        pltpu.VMEM((CH, C), jnp.float32),
        pltpu.VMEM((RPW,), jnp.int32),
        pltpu.VMEM((RPW,), jnp.float32),
        pltpu.VMEM((16, 16), jnp.float32),
        pltpu.SemaphoreType.DMA,
        pltpu.SemaphoreType.DMA,
    ],
    compiler_params=pltpu.CompilerParams(needs_layout_passes=False),
)


@jax.jit
def kernel(pred, y):
    return _run(pred, y.astype(jnp.int32))
